# baseline (speedup 1.0000x reference)
import functools

import jax
import jax.numpy as jnp
from jax import lax
from jax.experimental import pallas as pl
from jax.experimental.pallas import tpu as pltpu
from jax.experimental.pallas import tpu_sc as plsc

N_NODES = 10000
N_EDGES = 320000
D_IN = 128
D_OUT = 64
D_HYPER = 128
D_META = 64
SLOPE = 0.2

BLK = 400
OC = 8

NC = 2
NS = 16
NW = NC * NS
SUB = 64
ROWS_TOT = 5120
E_PAD = ROWS_TOT * SUB
ROWS_W = ROWS_TOT // NW
KSUB = 4
CHUNK = SUB * KSUB
NCHUNK = ROWS_W // KSUB
VEC = 16


def _hyper_body(x_ref, m_ref, wf_ref, g2_ref, wa_ref, aio_ref, ajo_ref,
                xt_ref, abc_ref):
    x = x_ref[...]
    xm = jnp.concatenate([x, m_ref[...]], axis=1)
    f = jnp.tanh(jnp.dot(xm, wf_ref[...], preferred_element_type=jnp.float32))
    parts = []
    ow = D_OUT // OC
    for oc in range(OC):
        g = jnp.dot(x, g2_ref[:, oc * ow * D_HYPER:(oc + 1) * ow * D_HYPER],
                    preferred_element_type=jnp.float32)
        g3 = g.reshape(BLK, ow, D_HYPER)
        parts.append((g3 * f[:, None, :]).sum(-1))
    xt = jnp.concatenate(parts, axis=1)
    ain = jnp.dot(f, wa_ref[...], preferred_element_type=jnp.float32)
    ai = (xt * aio_ref[...] * ain).sum(-1)
    aj = (xt * ajo_ref[...] * ain).sum(-1)
    s = ai + aj
    c = jnp.where(s > 0, s, s * SLOPE)
    z = jnp.zeros_like(ai)
    xt_ref[...] = xt
    abc_ref[...] = jnp.stack([ai, aj, c, z, z, z, z, z], axis=1)


def _run_hyper(x, meta_k, wfT, g2, waT, aio, ajo):
    grid = N_NODES // BLK
    return pl.pallas_call(
        _hyper_body,
        grid=(grid,),
        in_specs=[
            pl.BlockSpec((BLK, D_IN), lambda i: (i, 0)),
            pl.BlockSpec((BLK, D_META), lambda i: (i, 0)),
            pl.BlockSpec((D_IN + D_META, D_HYPER), lambda i: (0, 0)),
            pl.BlockSpec((D_IN, D_OUT * D_HYPER), lambda i: (0, 0)),
            pl.BlockSpec((D_HYPER, D_OUT), lambda i: (0, 0)),
            pl.BlockSpec((1, D_OUT), lambda i: (0, 0)),
            pl.BlockSpec((1, D_OUT), lambda i: (0, 0)),
        ],
        out_specs=[
            pl.BlockSpec((BLK, D_OUT), lambda i: (i, 0)),
            pl.BlockSpec((BLK, 8), lambda i: (i, 0)),
        ],
        out_shape=[
            jax.ShapeDtypeStruct((N_NODES, D_OUT), jnp.float32),
            jax.ShapeDtypeStruct((N_NODES, 8), jnp.float32),
        ],
    )(x, meta_k, wfT, g2, waT, aio, ajo)


def _splat16(vec, lane):
    idx = jnp.full((VEC,), lane, jnp.int32)
    dn = lax.GatherDimensionNumbers(offset_dims=(), collapsed_slice_dims=(0,),
                                    start_index_map=(0,))
    return lax.gather(vec, idx[:, None], dn, (1,),
                      mode=lax.GatherScatterMode.PROMISE_IN_BOUNDS)


def _sc_edge_body(src_h, dst_h, ai_h, aj_h, xt_h, zeros_h,
                  num_out, den_out,
                  ai_t, aj_t, den_t, sidx, didx, ebuf, rows,
                  gsem, asem, num_sp):
    cid = lax.axis_index("c")
    sid = lax.axis_index("s")
    wid = sid * NC + cid
    wbase = wid * ROWS_W
    pltpu.sync_copy(ai_h, ai_t)
    pltpu.sync_copy(aj_h, aj_t)

    zv = jnp.zeros((VEC,), jnp.float32)

    def zbody(i, carry):
        den_t[pl.ds(i * VEC, VEC)] = zv
        return carry

    lax.fori_loop(0, N_NODES // VEC, zbody, 0)

    @pl.when(sid == 0)
    def _():
        pltpu.sync_copy(zeros_h, num_sp)

    plsc.subcore_barrier()

    def stage_and_fire(jc, p):
        roff = wbase + jc * KSUB
        pltpu.sync_copy(src_h.at[pl.ds(roff, KSUB)], sidx.at[p])
        pltpu.sync_copy(dst_h.at[pl.ds(roff, KSUB)], didx.at[p])
        for b in range(KSUB):
            pltpu.async_copy(xt_h.at[sidx.at[p, b]], rows.at[p, b], gsem)

    def wait_gathers(p):
        for b in range(KSUB):
            pltpu.make_async_copy(xt_h.at[sidx.at[p, b]], rows.at[p, b],
                                  gsem).wait()

    def fire_adds(p):
        for b in range(KSUB):
            pltpu.async_copy(rows.at[p, b], num_sp.at[didx.at[p, b]],
                             asem, add=True)

    def wait_adds(p):
        for b in range(KSUB):
            pltpu.make_async_copy(rows.at[p, b], num_sp.at[didx.at[p, b]],
                                  asem).wait()

    def e_phase(p):
        for b in range(KSUB):
            def ebody(q, carry2, b=b):
                s16 = sidx[p, b, pl.ds(q * VEC, VEC)]
                d16 = didx[p, b, pl.ds(q * VEC, VEC)]
                aiv = plsc.load_gather(ai_t, [d16])
                ajd = plsc.load_gather(aj_t, [d16])
                ajv = plsc.load_gather(aj_t, [s16])
                al = aiv + ajv
                al = jnp.where(al > 0, al, al * SLOPE)
                cs = aiv + ajd
                cs = jnp.where(cs > 0, cs, cs * SLOPE)
                e = jnp.exp(jnp.minimum(al - cs, 60.0))
                e = jnp.where(s16 != d16, e, 0.0)
                plsc.addupdate_scatter(den_t, [d16], e)
                ebuf[p, pl.ds(b * SUB + q * VEC, VEC)] = e
                return carry2

            lax.fori_loop(0, SUB // VEC, ebody, 0)

    def scale(p):
        for b in range(KSUB):
            def rbody(g, carry2, b=b):
                ev = ebuf[p, pl.ds(b * SUB + g * VEC, VEC)]
                for t in range(VEC):
                    et = _splat16(ev, t)
                    rr = g * VEC + t
                    for cc in range(D_OUT // VEC):
                        sl = pl.ds(cc * VEC, VEC)
                        rows[p, b, rr, sl] = rows[p, b, rr, sl] * et
                return carry2

            lax.fori_loop(0, SUB // VEC, rbody, 0)

    stage_and_fire(0, 0)

    def piped(jj, carry):
        for p in (0, 1):
            jc = 2 * jj + p
            wait_gathers(p)
            e_phase(p)

            @pl.when(jc >= 1)
            def _():
                wait_adds(1 - p)

            scale(p)

            @pl.when(jc + 1 < NCHUNK)
            def _():
                stage_and_fire(jc + 1, 1 - p)

            fire_adds(p)
        return carry

    lax.fori_loop(0, NCHUNK // 2, piped, 0)
    wait_adds(1)

    pltpu.sync_copy(den_t, den_out.at[wid])
    plsc.subcore_barrier()

    @pl.when(sid == 0)
    def _():
        pltpu.sync_copy(num_sp, num_out.at[cid])


def _sc_edge_call(src2, dst2, ai, aj, xt, zeros):
    mesh = plsc.VectorSubcoreMesh(core_axis_name="c", subcore_axis_name="s")
    kfn = pl.kernel(
        _sc_edge_body,
        out_type=[
            jax.ShapeDtypeStruct((NC, N_NODES, D_OUT), jnp.float32),
            jax.ShapeDtypeStruct((NW, N_NODES), jnp.float32),
        ],
        mesh=mesh,
        scratch_types=[
            pltpu.VMEM((N_NODES,), jnp.float32),
            pltpu.VMEM((N_NODES,), jnp.float32),
            pltpu.VMEM((N_NODES,), jnp.float32),
            pltpu.VMEM((2, KSUB, SUB), jnp.int32),
            pltpu.VMEM((2, KSUB, SUB), jnp.int32),
            pltpu.VMEM((2, CHUNK), jnp.float32),
            pltpu.VMEM((2, KSUB, SUB, D_OUT), jnp.float32),
            pltpu.SemaphoreType.DMA,
            pltpu.SemaphoreType.DMA,
            pltpu.VMEM_SHARED((N_NODES, D_OUT), jnp.float32),
        ],
        compiler_params=pltpu.CompilerParams(needs_layout_passes=False,
                                             use_tc_tiling_on_sc=False),
    )
    return kfn(src2, dst2, ai, aj, xt, zeros)


def _finish_body(xt_ref, num_ref, den_ref, out_ref):
    num = num_ref[...]
    total = num[0] + num[1] + xt_ref[...]
    den = den_ref[...].sum(axis=1) + 1.0
    r = total / den[:, None]
    out_ref[...] = jnp.where(r > 0, r, jnp.exp(r) - 1.0)


def _run_finish(xt, num2, den32):
    grid = N_NODES // BLK
    return pl.pallas_call(
        _finish_body,
        grid=(grid,),
        in_specs=[
            pl.BlockSpec((BLK, D_OUT), lambda i: (i, 0)),
            pl.BlockSpec((2, BLK, D_OUT), lambda i: (0, i, 0)),
            pl.BlockSpec((BLK, NW), lambda i: (i, 0)),
        ],
        out_specs=pl.BlockSpec((BLK, D_OUT), lambda i: (i, 0)),
        out_shape=jax.ShapeDtypeStruct((N_NODES, D_OUT), jnp.float32),
    )(xt, num2, den32)


def kernel(meta_k, node_emb, W_f, W_hyper, W_att, att_i_o, att_j_o, edge_index):
    x = node_emb
    wfT = W_f.T
    g2 = (W_hyper.reshape(D_OUT, D_IN, D_HYPER)
          .transpose(1, 0, 2).reshape(D_IN, D_OUT * D_HYPER))
    waT = W_att[:D_OUT].T
    aio = att_i_o.reshape(1, D_OUT)
    ajo = att_j_o.reshape(1, D_OUT)
    xt, abc = _run_hyper(x, meta_k, wfT, g2, waT, aio, ajo)
    ai, aj = abc[:, 0], abc[:, 1]
    pad = jnp.zeros((2, E_PAD - N_EDGES), jnp.int32)
    ei = jnp.concatenate([edge_index, pad], axis=1)
    src2 = ei[0].reshape(ROWS_TOT, SUB)
    dst2 = ei[1].reshape(ROWS_TOT, SUB)
    zeros = jnp.zeros((N_NODES, D_OUT), jnp.float32)
    num2, den32 = _sc_edge_call(src2, dst2, ai, aj, xt, zeros)
    return _run_finish(xt, num2, den32.T)


# speedup vs baseline: 1.1446x; 1.1446x over previous
import functools

import jax
import jax.numpy as jnp
from jax import lax
from jax.experimental import pallas as pl
from jax.experimental.pallas import tpu as pltpu
from jax.experimental.pallas import tpu_sc as plsc

N_NODES = 10000
N_EDGES = 320000
D_IN = 128
D_OUT = 64
D_HYPER = 128
D_META = 64
SLOPE = 0.2

BLK = 400
OC = 8

NC = 2
NS = 16
NW = NC * NS
SUB = 128
ROWS_TOT = 2560
E_PAD = ROWS_TOT * SUB
KSUB = 2
CHUNK = SUB * KSUB
R_C0 = 100
R_C1 = 60
NCH0 = R_C0 // KSUB
NCH1 = R_C1 // KSUB
VEC = 16


def _hyper_body(x_ref, m_ref, wf_ref, g2_ref, wa_ref, aio_ref, ajo_ref,
                xt_ref, abc_ref):
    x = x_ref[...]
    xm = jnp.concatenate([x, m_ref[...]], axis=1)
    f = jnp.tanh(jnp.dot(xm, wf_ref[...], preferred_element_type=jnp.float32))
    fb = f.astype(jnp.bfloat16)
    parts = []
    ow = D_OUT // OC
    for oc in range(OC):
        g = jnp.dot(x, g2_ref[:, oc * ow * D_HYPER:(oc + 1) * ow * D_HYPER],
                    preferred_element_type=jnp.float32)
        g3 = g.reshape(BLK, ow, D_HYPER)
        parts.append((g3 * f[:, None, :]).sum(-1))
    xt = jnp.concatenate(parts, axis=1)
    ain = jnp.dot(fb, wa_ref[...], preferred_element_type=jnp.float32)
    ai = (xt * aio_ref[...] * ain).sum(-1)
    aj = (xt * ajo_ref[...] * ain).sum(-1)
    s = ai + aj
    c = jnp.where(s > 0, s, s * SLOPE)
    z = jnp.zeros_like(ai)
    xt_ref[...] = xt
    abc_ref[...] = jnp.stack([ai, aj, c, z, z, z, z, z], axis=1)


def _run_hyper(x, meta_k, wfT, g2, waT, aio, ajo):
    grid = N_NODES // BLK
    return pl.pallas_call(
        _hyper_body,
        grid=(grid,),
        in_specs=[
            pl.BlockSpec((BLK, D_IN), lambda i: (i, 0)),
            pl.BlockSpec((BLK, D_META), lambda i: (i, 0)),

            pl.BlockSpec((D_IN + D_META, D_HYPER), lambda i: (0, 0)),
            pl.BlockSpec((D_IN, D_OUT * D_HYPER), lambda i: (0, 0)),
            pl.BlockSpec((D_HYPER, D_OUT), lambda i: (0, 0)),
            pl.BlockSpec((1, D_OUT), lambda i: (0, 0)),
            pl.BlockSpec((1, D_OUT), lambda i: (0, 0)),
        ],
        out_specs=[
            pl.BlockSpec((BLK, D_OUT), lambda i: (i, 0)),
            pl.BlockSpec((BLK, 8), lambda i: (i, 0)),
        ],
        out_shape=[
            jax.ShapeDtypeStruct((N_NODES, D_OUT), jnp.float32),
            jax.ShapeDtypeStruct((N_NODES, 8), jnp.float32),
        ],
    )(x, meta_k, wfT, g2, waT, aio, ajo)


def _splat16(vec, lane):
    idx = jnp.full((VEC,), lane, jnp.int32)
    dn = lax.GatherDimensionNumbers(offset_dims=(), collapsed_slice_dims=(0,),
                                    start_index_map=(0,))
    return lax.gather(vec, idx[:, None], dn, (1,),
                      mode=lax.GatherScatterMode.PROMISE_IN_BOUNDS)


def _sc_edge_body(sd_h, ai_h, aj_h, xt_h, zeros_h,
                  num_out, den_out,
                  ai_t, aj_t, den_t, sdix, ebuf, rows,
                  gsem, asem, isem, num_sp):
    cid = lax.axis_index("c")
    sid = lax.axis_index("s")
    wid = sid * NC + cid
    wbase = jnp.where(cid == 0, sid * R_C0, NS * R_C0 + sid * R_C1)
    nchunk = jnp.where(cid == 0, NCH0, NCH1)
    pltpu.sync_copy(ai_h, ai_t)
    pltpu.sync_copy(aj_h, aj_t)

    zv = jnp.zeros((VEC,), jnp.float32)

    def zbody(i, carry):
        den_t[pl.ds(i * VEC, VEC)] = zv
        return carry

    lax.fori_loop(0, N_NODES // VEC, zbody, 0)

    @pl.when(sid == 0)
    def _():
        pltpu.sync_copy(zeros_h, num_sp)

    plsc.subcore_barrier()

    def fire_idx(jc, p):
        pltpu.async_copy(sd_h.at[pl.ds(wbase + jc * KSUB, KSUB)],
                         sdix.at[p], isem)

    def wait_idx(jc, p):
        pltpu.make_async_copy(sd_h.at[pl.ds(wbase + jc * KSUB, KSUB)],
                              sdix.at[p], isem).wait()

    def fire_gathers(p):
        for b in range(KSUB):
            pltpu.async_copy(xt_h.at[sdix.at[p, b, 0]], rows.at[p, b], gsem)

    def wait_gathers(p):
        for b in range(KSUB):
            pltpu.make_async_copy(xt_h.at[sdix.at[p, b, 0]], rows.at[p, b],
                                  gsem).wait()

    def fire_adds(p):
        for b in range(KSUB):
            pltpu.async_copy(rows.at[p, b], num_sp.at[sdix.at[p, b, 1]],
                             asem, add=True)

    def wait_adds(p):
        for b in range(KSUB):
            pltpu.make_async_copy(rows.at[p, b], num_sp.at[sdix.at[p, b, 1]],
                                  asem).wait()

    def e_phase(p):
        for b in range(KSUB):
            def ebody(q, carry2, b=b):
                s16 = sdix[p, b, 0, pl.ds(q * VEC, VEC)]
                d16 = sdix[p, b, 1, pl.ds(q * VEC, VEC)]
                aiv = plsc.load_gather(ai_t, [d16])
                ajd = plsc.load_gather(aj_t, [d16])
                ajv = plsc.load_gather(aj_t, [s16])
                al = aiv + ajv
                al = jnp.where(al > 0, al, al * SLOPE)
                cs = aiv + ajd
                cs = jnp.where(cs > 0, cs, cs * SLOPE)
                e = jnp.exp(jnp.minimum(al - cs, 60.0))
                e = jnp.where(s16 != d16, e, 0.0)
                plsc.addupdate_scatter(den_t, [d16], e)
                ebuf[p, pl.ds(b * SUB + q * VEC, VEC)] = e
                return carry2

            lax.fori_loop(0, SUB // VEC, ebody, 0)

    def scale(p):
        for b in range(KSUB):
            def rbody(g, carry2, b=b):
                ev = ebuf[p, pl.ds(b * SUB + g * VEC, VEC)]
                for t in range(VEC):
                    et = _splat16(ev, t)
                    rr = g * VEC + t
                    for cc in range(D_OUT // VEC):
                        sl = pl.ds(cc * VEC, VEC)
                        rows[p, b, rr, sl] = rows[p, b, rr, sl] * et
                return carry2

            lax.fori_loop(0, SUB // VEC, rbody, 0)

    pltpu.sync_copy(sd_h.at[pl.ds(wbase, KSUB)], sdix.at[0])
    fire_gathers(0)

    def piped(jj, carry):
        for p in (0, 1):
            jc = 2 * jj + p
            wait_gathers(p)
            e_phase(p)

            @pl.when(jc >= 1)
            def _():
                wait_adds(1 - p)

            @pl.when(jc + 1 < nchunk)
            def _():
                fire_idx(jc + 1, 1 - p)

            scale(p)

            @pl.when(jc + 1 < nchunk)
            def _():
                wait_idx(jc + 1, 1 - p)
                fire_gathers(1 - p)

            fire_adds(p)
        return carry

    lax.fori_loop(0, nchunk // 2, piped, 0)
    wait_adds(1)

    pltpu.sync_copy(den_t, den_out.at[wid])
    plsc.subcore_barrier()

    @pl.when(sid == 0)
    def _():
        pltpu.sync_copy(num_sp, num_out.at[cid])


def _sc_edge_call(sd, ai, aj, xt, zeros):
    mesh = plsc.VectorSubcoreMesh(core_axis_name="c", subcore_axis_name="s")
    kfn = pl.kernel(
        _sc_edge_body,
        out_type=[
            jax.ShapeDtypeStruct((NC, N_NODES, D_OUT), jnp.float32),
            jax.ShapeDtypeStruct((NW, N_NODES), jnp.float32),
        ],
        mesh=mesh,
        scratch_types=[
            pltpu.VMEM((N_NODES,), jnp.float32),
            pltpu.VMEM((N_NODES,), jnp.float32),
            pltpu.VMEM((N_NODES,), jnp.float32),
            pltpu.VMEM((2, KSUB, 2, SUB), jnp.int32),
            pltpu.VMEM((2, CHUNK), jnp.float32),
            pltpu.VMEM((2, KSUB, SUB, D_OUT), jnp.float32),
            pltpu.SemaphoreType.DMA,
            pltpu.SemaphoreType.DMA,
            pltpu.SemaphoreType.DMA,
            pltpu.VMEM_SHARED((N_NODES, D_OUT), jnp.float32),
        ],
        compiler_params=pltpu.CompilerParams(needs_layout_passes=False,
                                             use_tc_tiling_on_sc=False),
    )
    return kfn(sd, ai, aj, xt, zeros)


def _finish_body(xt_ref, num_ref, den_ref, out_ref):
    num = num_ref[...]
    total = num[0] + num[1] + xt_ref[...]
    den = den_ref[...].sum(axis=1) + 1.0
    r = total / den[:, None]
    out_ref[...] = jnp.where(r > 0, r, jnp.exp(r) - 1.0)


def _run_finish(xt, num2, den32):
    grid = N_NODES // BLK
    return pl.pallas_call(
        _finish_body,
        grid=(grid,),
        in_specs=[
            pl.BlockSpec((BLK, D_OUT), lambda i: (i, 0)),
            pl.BlockSpec((2, BLK, D_OUT), lambda i: (0, i, 0)),
            pl.BlockSpec((BLK, NW), lambda i: (i, 0)),
        ],
        out_specs=pl.BlockSpec((BLK, D_OUT), lambda i: (i, 0)),
        out_shape=jax.ShapeDtypeStruct((N_NODES, D_OUT), jnp.float32),
    )(xt, num2, den32)


def kernel(meta_k, node_emb, W_f, W_hyper, W_att, att_i_o, att_j_o, edge_index):
    x = node_emb.astype(jnp.bfloat16)
    mb = meta_k.astype(jnp.bfloat16)
    wfT = W_f.T.astype(jnp.bfloat16)
    g2 = (W_hyper.reshape(D_OUT, D_IN, D_HYPER)
          .transpose(1, 0, 2).reshape(D_IN, D_OUT * D_HYPER)
          .astype(jnp.bfloat16))
    waT = W_att[:D_OUT].T.astype(jnp.bfloat16)
    aio = att_i_o.reshape(1, D_OUT)
    ajo = att_j_o.reshape(1, D_OUT)
    xt, abc = _run_hyper(x, mb, wfT, g2, waT, aio, ajo)
    ai, aj = abc[:, 0], abc[:, 1]
    pad = jnp.zeros((2, E_PAD - N_EDGES), jnp.int32)
    ei = jnp.concatenate([edge_index, pad], axis=1)
    sd = ei.reshape(2, ROWS_TOT, SUB).transpose(1, 0, 2)
    zeros = jnp.zeros((N_NODES, D_OUT), jnp.float32)
    num2, den32 = _sc_edge_call(sd, ai, aj, xt, zeros)
    return _run_finish(xt, num2, den32.T)


# speedup vs baseline: 1.3824x; 1.2077x over previous
import functools

import jax
import jax.numpy as jnp
from jax import lax
from jax.experimental import pallas as pl
from jax.experimental.pallas import tpu as pltpu
from jax.experimental.pallas import tpu_sc as plsc

N_NODES = 10000
N_EDGES = 320000
D_IN = 128
D_OUT = 64
D_HYPER = 128
D_META = 64
SLOPE = 0.2

BLK = 400
OC = 8

NC = 2
NS = 16
NW = NC * NS
SUB = 128
ROWS_TOT = 2560
E_PAD = ROWS_TOT * SUB
NBUF = 4
R_C0 = 100
R_C1 = 60
VEC = 16


def _hyper_body(x_ref, m_ref, wf_ref, g2_ref, wa_ref, aio_ref, ajo_ref,
                xt_ref, abc_ref):
    x = x_ref[...]
    xm = jnp.concatenate([x, m_ref[...]], axis=1)
    f = jnp.tanh(jnp.dot(xm, wf_ref[...], preferred_element_type=jnp.float32))
    fb = f.astype(jnp.bfloat16)
    parts = []
    ow = D_OUT // OC
    for oc in range(OC):
        g = jnp.dot(x, g2_ref[:, oc * ow * D_HYPER:(oc + 1) * ow * D_HYPER],
                    preferred_element_type=jnp.float32)
        g3 = g.reshape(BLK, ow, D_HYPER)
        parts.append((g3 * f[:, None, :]).sum(-1))
    xt = jnp.concatenate(parts, axis=1)
    ain = jnp.dot(fb, wa_ref[...], preferred_element_type=jnp.float32)
    ai = (xt * aio_ref[...] * ain).sum(-1)
    aj = (xt * ajo_ref[...] * ain).sum(-1)
    s = ai + aj
    c = jnp.where(s > 0, s, s * SLOPE)
    z = jnp.zeros_like(ai)
    xt_ref[...] = xt
    abc_ref[...] = jnp.stack([ai, aj, c, z, z, z, z, z], axis=1)


def _run_hyper(x, meta_k, wfT, g2, waT, aio, ajo):
    grid = N_NODES // BLK
    return pl.pallas_call(
        _hyper_body,
        grid=(grid,),
        in_specs=[
            pl.BlockSpec((BLK, D_IN), lambda i: (i, 0)),
            pl.BlockSpec((BLK, D_META), lambda i: (i, 0)),

            pl.BlockSpec((D_IN + D_META, D_HYPER), lambda i: (0, 0)),
            pl.BlockSpec((D_IN, D_OUT * D_HYPER), lambda i: (0, 0)),
            pl.BlockSpec((D_HYPER, D_OUT), lambda i: (0, 0)),
            pl.BlockSpec((1, D_OUT), lambda i: (0, 0)),
            pl.BlockSpec((1, D_OUT), lambda i: (0, 0)),
        ],
        out_specs=[
            pl.BlockSpec((BLK, D_OUT), lambda i: (i, 0)),
            pl.BlockSpec((BLK, 8), lambda i: (i, 0)),
        ],
        out_shape=[
            jax.ShapeDtypeStruct((N_NODES, D_OUT), jnp.float32),
            jax.ShapeDtypeStruct((N_NODES, 8), jnp.float32),
        ],
    )(x, meta_k, wfT, g2, waT, aio, ajo)


def _splat16(vec, lane):
    idx = jnp.full((VEC,), lane, jnp.int32)
    dn = lax.GatherDimensionNumbers(offset_dims=(), collapsed_slice_dims=(0,),
                                    start_index_map=(0,))
    return lax.gather(vec, idx[:, None], dn, (1,),
                      mode=lax.GatherScatterMode.PROMISE_IN_BOUNDS)


def _sc_edge_body(sd_h, ai_h, aj_h, xt_h, zeros_h,
                  num_out, den_out,
                  ai_t, aj_t, den_t, sdix, ebuf, rows,
                  gsem, asem, isem, num_sp):
    cid = lax.axis_index("c")
    sid = lax.axis_index("s")
    wid = sid * NC + cid
    wbase = jnp.where(cid == 0, sid * R_C0, NS * R_C0 + sid * R_C1)
    nchunk = jnp.where(cid == 0, R_C0, R_C1)
    pltpu.sync_copy(ai_h, ai_t)
    pltpu.sync_copy(aj_h, aj_t)

    zv = jnp.zeros((VEC,), jnp.float32)

    def zbody(i, carry):
        den_t[pl.ds(i * VEC, VEC)] = zv
        return carry

    lax.fori_loop(0, N_NODES // VEC, zbody, 0)

    @pl.when(sid == 0)
    def _():
        pltpu.sync_copy(zeros_h, num_sp)

    plsc.subcore_barrier()

    def fire_idx(jc, s):
        pltpu.async_copy(sd_h.at[wbase + jc], sdix.at[s], isem)

    def wait_idx(jc, s):
        pltpu.make_async_copy(sd_h.at[wbase + jc], sdix.at[s], isem).wait()

    def fire_gathers(s):
        pltpu.async_copy(xt_h.at[sdix.at[s, 0]], rows.at[s], gsem)

    def wait_gathers(s):
        pltpu.make_async_copy(xt_h.at[sdix.at[s, 0]], rows.at[s],
                              gsem).wait()

    def fire_adds(s):
        pltpu.async_copy(rows.at[s], num_sp.at[sdix.at[s, 1]], asem,
                         add=True)

    def wait_adds(s):
        pltpu.make_async_copy(rows.at[s], num_sp.at[sdix.at[s, 1]],
                              asem).wait()

    def e_phase(s):
        def ebody(q, carry2):
            s16 = sdix[s, 0, pl.ds(q * VEC, VEC)]
            d16 = sdix[s, 1, pl.ds(q * VEC, VEC)]
            aiv = plsc.load_gather(ai_t, [d16])
            ajd = plsc.load_gather(aj_t, [d16])
            ajv = plsc.load_gather(aj_t, [s16])
            al = aiv + ajv
            al = jnp.where(al > 0, al, al * SLOPE)
            cs = aiv + ajd
            cs = jnp.where(cs > 0, cs, cs * SLOPE)
            e = jnp.exp(jnp.minimum(al - cs, 60.0))
            e = jnp.where(s16 != d16, e, 0.0)
            plsc.addupdate_scatter(den_t, [d16], e)
            ebuf[pl.ds(q * VEC, VEC)] = e
            return carry2

        lax.fori_loop(0, SUB // VEC, ebody, 0)

    def scale(s):
        def rbody(g, carry2):
            ev = ebuf[pl.ds(g * VEC, VEC)]
            for t in range(VEC):
                et = _splat16(ev, t)
                rr = g * VEC + t
                for cc in range(D_OUT // VEC):
                    sl = pl.ds(cc * VEC, VEC)
                    rows[s, rr, sl] = rows[s, rr, sl] * et
            return carry2

        lax.fori_loop(0, SUB // VEC, rbody, 0)

    pltpu.sync_copy(sd_h.at[wbase], sdix.at[0])
    fire_gathers(0)
    fire_idx(1, 1)

    def piped(ii, carry):
        for u in range(NBUF):
            jc = NBUF * ii + u
            s = u

            @pl.when(jc >= 2)
            def _():
                wait_adds((s + 2) % NBUF)

            @pl.when(jc + 1 < nchunk)
            def _():
                wait_idx(jc + 1, (s + 1) % NBUF)
                fire_gathers((s + 1) % NBUF)

            @pl.when(jc + 2 < nchunk)
            def _():
                fire_idx(jc + 2, (s + 2) % NBUF)

            wait_gathers(s)
            e_phase(s)
            scale(s)
            fire_adds(s)
        return carry

    lax.fori_loop(0, nchunk // NBUF, piped, 0)
    wait_adds(2)
    wait_adds(3)

    pltpu.sync_copy(den_t, den_out.at[wid])
    plsc.subcore_barrier()

    @pl.when(sid == 0)
    def _():
        pltpu.sync_copy(num_sp, num_out.at[cid])


def _sc_edge_call(sd, ai, aj, xt, zeros):
    mesh = plsc.VectorSubcoreMesh(core_axis_name="c", subcore_axis_name="s")
    kfn = pl.kernel(
        _sc_edge_body,
        out_type=[
            jax.ShapeDtypeStruct((NC, N_NODES, D_OUT), jnp.float32),
            jax.ShapeDtypeStruct((NW, N_NODES), jnp.float32),
        ],
        mesh=mesh,
        scratch_types=[
            pltpu.VMEM((N_NODES,), jnp.float32),
            pltpu.VMEM((N_NODES,), jnp.float32),
            pltpu.VMEM((N_NODES,), jnp.float32),
            pltpu.VMEM((NBUF, 2, SUB), jnp.int32),
            pltpu.VMEM((SUB,), jnp.float32),
            pltpu.VMEM((NBUF, SUB, D_OUT), jnp.float32),
            pltpu.SemaphoreType.DMA,
            pltpu.SemaphoreType.DMA,
            pltpu.SemaphoreType.DMA,
            pltpu.VMEM_SHARED((N_NODES, D_OUT), jnp.float32),
        ],
        compiler_params=pltpu.CompilerParams(needs_layout_passes=False,
                                             use_tc_tiling_on_sc=False),
    )
    return kfn(sd, ai, aj, xt, zeros)


def _finish_body(xt_ref, num_ref, den_ref, out_ref):
    num = num_ref[...]
    total = num[0] + num[1] + xt_ref[...]
    den = den_ref[...].sum(axis=1) + 1.0
    r = total / den[:, None]
    out_ref[...] = jnp.where(r > 0, r, jnp.exp(r) - 1.0)


def _run_finish(xt, num2, den32):
    grid = N_NODES // BLK
    return pl.pallas_call(
        _finish_body,
        grid=(grid,),
        in_specs=[
            pl.BlockSpec((BLK, D_OUT), lambda i: (i, 0)),
            pl.BlockSpec((2, BLK, D_OUT), lambda i: (0, i, 0)),
            pl.BlockSpec((BLK, NW), lambda i: (i, 0)),
        ],
        out_specs=pl.BlockSpec((BLK, D_OUT), lambda i: (i, 0)),
        out_shape=jax.ShapeDtypeStruct((N_NODES, D_OUT), jnp.float32),
    )(xt, num2, den32)


def kernel(meta_k, node_emb, W_f, W_hyper, W_att, att_i_o, att_j_o, edge_index):
    x = node_emb.astype(jnp.bfloat16)
    mb = meta_k.astype(jnp.bfloat16)
    wfT = W_f.T.astype(jnp.bfloat16)
    g2 = (W_hyper.reshape(D_OUT, D_IN, D_HYPER)
          .transpose(1, 0, 2).reshape(D_IN, D_OUT * D_HYPER)
          .astype(jnp.bfloat16))
    waT = W_att[:D_OUT].T.astype(jnp.bfloat16)
    aio = att_i_o.reshape(1, D_OUT)
    ajo = att_j_o.reshape(1, D_OUT)
    xt, abc = _run_hyper(x, mb, wfT, g2, waT, aio, ajo)
    ai, aj = abc[:, 0], abc[:, 1]
    pad = jnp.zeros((2, E_PAD - N_EDGES), jnp.int32)
    ei = jnp.concatenate([edge_index, pad], axis=1)
    sd = ei.reshape(2, ROWS_TOT, SUB).transpose(1, 0, 2)
    zeros = jnp.zeros((N_NODES, D_OUT), jnp.float32)
    num2, den32 = _sc_edge_call(sd, ai, aj, xt, zeros)
    return _run_finish(xt, num2, den32.T)


# speedup vs baseline: 1.4763x; 1.0680x over previous
import functools

import jax
import jax.numpy as jnp
from jax import lax
from jax.experimental import pallas as pl
from jax.experimental.pallas import tpu as pltpu
from jax.experimental.pallas import tpu_sc as plsc

N_NODES = 10000
N_EDGES = 320000
D_IN = 128
D_OUT = 64
D_HYPER = 128
D_META = 64
SLOPE = 0.2

BLK = 400
OC = 8

NC = 2
NS = 16
NW = NC * NS
SUB = 128
ROWS_TOT = 2560
E_PAD = ROWS_TOT * SUB
NBUF = 4
R_C0 = 104
R_C1 = 56
VEC = 16


def _hyper_body(x_ref, m_ref, wf_ref, g2_ref, wa_ref, aio_ref, ajo_ref,
                xt_ref, abc_ref):
    x = x_ref[...]
    xm = jnp.concatenate([x, m_ref[...]], axis=1)
    f = jnp.tanh(jnp.dot(xm, wf_ref[...], preferred_element_type=jnp.float32))
    fb = f.astype(jnp.bfloat16)
    parts = []
    ow = D_OUT // OC
    ft = jnp.concatenate([f] * ow, axis=1)
    for oc in range(OC):
        g = jnp.dot(x, g2_ref[:, oc * ow * D_HYPER:(oc + 1) * ow * D_HYPER],
                    preferred_element_type=jnp.float32)
        m = g * ft
        for o in range(ow):
            sl = m[:, o * D_HYPER:(o + 1) * D_HYPER].sum(-1)
            parts.append(sl.reshape(BLK, 1))
    xt = jnp.concatenate(parts, axis=1)
    ain = jnp.dot(fb, wa_ref[...], preferred_element_type=jnp.float32)
    ai = (xt * aio_ref[...] * ain).sum(-1)
    aj = (xt * ajo_ref[...] * ain).sum(-1)
    s = ai + aj
    c = jnp.where(s > 0, s, s * SLOPE)
    z = jnp.zeros_like(ai)
    xt_ref[...] = xt
    abc_ref[...] = jnp.stack([ai, aj, c, z, z, z, z, z], axis=1)


def _run_hyper(x, meta_k, wfT, g2, waT, aio, ajo):
    grid = N_NODES // BLK
    return pl.pallas_call(
        _hyper_body,
        grid=(grid,),
        in_specs=[
            pl.BlockSpec((BLK, D_IN), lambda i: (i, 0)),
            pl.BlockSpec((BLK, D_META), lambda i: (i, 0)),

            pl.BlockSpec((D_IN + D_META, D_HYPER), lambda i: (0, 0)),
            pl.BlockSpec((D_IN, D_OUT * D_HYPER), lambda i: (0, 0)),
            pl.BlockSpec((D_HYPER, D_OUT), lambda i: (0, 0)),
            pl.BlockSpec((1, D_OUT), lambda i: (0, 0)),
            pl.BlockSpec((1, D_OUT), lambda i: (0, 0)),
        ],
        out_specs=[
            pl.BlockSpec((BLK, D_OUT), lambda i: (i, 0)),
            pl.BlockSpec((BLK, 8), lambda i: (i, 0)),
        ],
        out_shape=[
            jax.ShapeDtypeStruct((N_NODES, D_OUT), jnp.float32),
            jax.ShapeDtypeStruct((N_NODES, 8), jnp.float32),
        ],
    )(x, meta_k, wfT, g2, waT, aio, ajo)


def _splat16(vec, lane):
    idx = jnp.full((VEC,), lane, jnp.int32)
    dn = lax.GatherDimensionNumbers(offset_dims=(), collapsed_slice_dims=(0,),
                                    start_index_map=(0,))
    return lax.gather(vec, idx[:, None], dn, (1,),
                      mode=lax.GatherScatterMode.PROMISE_IN_BOUNDS)


def _sc_edge_body(sd_h, ai_h, aj_h, xt_h, zeros_h,
                  num_out, den_out,
                  ai_t, aj_t, den_t, sdix, ebuf, rows,
                  gsem, asem, isem, num_sp):
    cid = lax.axis_index("c")
    sid = lax.axis_index("s")
    wid = sid * NC + cid
    wbase = jnp.where(cid == 0, sid * R_C0, NS * R_C0 + sid * R_C1)
    nchunk = jnp.where(cid == 0, R_C0, R_C1)
    pltpu.sync_copy(ai_h, ai_t)
    pltpu.sync_copy(aj_h, aj_t)

    zv = jnp.zeros((VEC,), jnp.float32)

    def zbody(i, carry):
        den_t[pl.ds(i * VEC, VEC)] = zv
        return carry

    lax.fori_loop(0, N_NODES // VEC, zbody, 0)

    @pl.when(sid == 0)
    def _():
        pltpu.sync_copy(zeros_h, num_sp)

    plsc.subcore_barrier()

    def fire_idx(jc, s):
        pltpu.async_copy(sd_h.at[wbase + jc], sdix.at[s], isem)

    def wait_idx(jc, s):
        pltpu.make_async_copy(sd_h.at[wbase + jc], sdix.at[s], isem).wait()

    def fire_gathers(s):
        pltpu.async_copy(xt_h.at[sdix.at[s, 0]], rows.at[s], gsem)

    def wait_gathers(s):
        pltpu.make_async_copy(xt_h.at[sdix.at[s, 0]], rows.at[s],
                              gsem).wait()

    def fire_adds(s):
        pltpu.async_copy(rows.at[s], num_sp.at[sdix.at[s, 1]], asem,
                         add=True)

    def wait_adds(s):
        pltpu.make_async_copy(rows.at[s], num_sp.at[sdix.at[s, 1]],
                              asem).wait()

    def e_phase(s):
        def ebody(q, carry2):
            s16 = sdix[s, 0, pl.ds(q * VEC, VEC)]
            d16 = sdix[s, 1, pl.ds(q * VEC, VEC)]
            aiv = plsc.load_gather(ai_t, [d16])
            ajd = plsc.load_gather(aj_t, [d16])
            ajv = plsc.load_gather(aj_t, [s16])
            al = aiv + ajv
            al = jnp.where(al > 0, al, al * SLOPE)
            cs = aiv + ajd
            cs = jnp.where(cs > 0, cs, cs * SLOPE)
            e = jnp.exp(jnp.minimum(al - cs, 60.0))
            e = jnp.where(s16 != d16, e, 0.0)
            plsc.addupdate_scatter(den_t, [d16], e)
            ebuf[pl.ds(q * VEC, VEC)] = e
            return carry2

        lax.fori_loop(0, SUB // VEC, ebody, 0)

    def scale(s):
        def rbody(g, carry2):
            ev = ebuf[pl.ds(g * VEC, VEC)]
            for t in range(VEC):
                et = _splat16(ev, t)
                rr = g * VEC + t
                for cc in range(D_OUT // VEC):
                    sl = pl.ds(cc * VEC, VEC)
                    rows[s, rr, sl] = rows[s, rr, sl] * et
            return carry2

        lax.fori_loop(0, SUB // VEC, rbody, 0)

    pltpu.sync_copy(sd_h.at[wbase], sdix.at[0])
    fire_gathers(0)
    fire_idx(1, 1)

    def piped(ii, carry):
        for u in range(NBUF):
            jc = NBUF * ii + u
            s = u

            @pl.when(jc >= 2)
            def _():
                wait_adds((s + 2) % NBUF)

            @pl.when(jc + 1 < nchunk)
            def _():
                wait_idx(jc + 1, (s + 1) % NBUF)
                fire_gathers((s + 1) % NBUF)

            @pl.when(jc + 2 < nchunk)
            def _():
                fire_idx(jc + 2, (s + 2) % NBUF)

            wait_gathers(s)
            e_phase(s)
            scale(s)
            fire_adds(s)
        return carry

    lax.fori_loop(0, nchunk // NBUF, piped, 0)
    wait_adds(2)
    wait_adds(3)

    pltpu.sync_copy(den_t, den_out.at[wid])
    plsc.subcore_barrier()

    @pl.when(sid == 0)
    def _():
        pltpu.sync_copy(num_sp, num_out.at[cid])


def _sc_edge_call(sd, ai, aj, xt, zeros):
    mesh = plsc.VectorSubcoreMesh(core_axis_name="c", subcore_axis_name="s")
    kfn = pl.kernel(
        _sc_edge_body,
        out_type=[
            jax.ShapeDtypeStruct((NC, N_NODES, D_OUT), jnp.float32),
            jax.ShapeDtypeStruct((NW, N_NODES), jnp.float32),
        ],
        mesh=mesh,
        scratch_types=[
            pltpu.VMEM((N_NODES,), jnp.float32),
            pltpu.VMEM((N_NODES,), jnp.float32),
            pltpu.VMEM((N_NODES,), jnp.float32),
            pltpu.VMEM((NBUF, 2, SUB), jnp.int32),
            pltpu.VMEM((SUB,), jnp.float32),
            pltpu.VMEM((NBUF, SUB, D_OUT), jnp.float32),
            pltpu.SemaphoreType.DMA,
            pltpu.SemaphoreType.DMA,
            pltpu.SemaphoreType.DMA,
            pltpu.VMEM_SHARED((N_NODES, D_OUT), jnp.float32),
        ],
        compiler_params=pltpu.CompilerParams(needs_layout_passes=False,
                                             use_tc_tiling_on_sc=False),
    )
    return kfn(sd, ai, aj, xt, zeros)


def _finish_body(xt_ref, num_ref, den_ref, out_ref):
    num = num_ref[...]
    total = num[0] + num[1] + xt_ref[...]
    den = den_ref[...].sum(axis=1) + 1.0
    r = total / den[:, None]
    out_ref[...] = jnp.where(r > 0, r, jnp.exp(r) - 1.0)


def _run_finish(xt, num2, den32):
    grid = N_NODES // BLK
    return pl.pallas_call(
        _finish_body,
        grid=(grid,),
        in_specs=[
            pl.BlockSpec((BLK, D_OUT), lambda i: (i, 0)),
            pl.BlockSpec((2, BLK, D_OUT), lambda i: (0, i, 0)),
            pl.BlockSpec((BLK, NW), lambda i: (i, 0)),
        ],
        out_specs=pl.BlockSpec((BLK, D_OUT), lambda i: (i, 0)),
        out_shape=jax.ShapeDtypeStruct((N_NODES, D_OUT), jnp.float32),
    )(xt, num2, den32)


def kernel(meta_k, node_emb, W_f, W_hyper, W_att, att_i_o, att_j_o, edge_index):
    x = node_emb.astype(jnp.bfloat16)
    mb = meta_k.astype(jnp.bfloat16)
    wfT = W_f.T.astype(jnp.bfloat16)
    g2 = (W_hyper.reshape(D_OUT, D_IN, D_HYPER)
          .transpose(1, 0, 2).reshape(D_IN, D_OUT * D_HYPER)
          .astype(jnp.bfloat16))
    waT = W_att[:D_OUT].T.astype(jnp.bfloat16)
    aio = att_i_o.reshape(1, D_OUT)
    ajo = att_j_o.reshape(1, D_OUT)
    xt, abc = _run_hyper(x, mb, wfT, g2, waT, aio, ajo)
    ai, aj = abc[:, 0], abc[:, 1]
    pad = jnp.zeros((2, E_PAD - N_EDGES), jnp.int32)
    ei = jnp.concatenate([edge_index, pad], axis=1)
    sd = ei.reshape(2, ROWS_TOT, SUB).transpose(1, 0, 2)
    zeros = jnp.zeros((N_NODES, D_OUT), jnp.float32)
    num2, den32 = _sc_edge_call(sd, ai, aj, xt, zeros)
    return _run_finish(xt, num2, den32.T)
